# baseline (speedup 1.0000x reference)
import jax
import jax.numpy as jnp
from jax import lax
from jax.experimental import pallas as pl
from jax.experimental.pallas import tpu as pltpu
from jax.experimental.pallas import tpu_sc as plsc

NC = 2
NS = 16
L = 16
B = 16384

N_HI = 608
N_LO = 416
HI_CORE = 1
CH_HI = (208, 208, 192)
CH_LO = (208, 208)

_P5 = (0.029808765243552946, -0.12995719765850117, 0.2838231830655296,
       -0.48969909032090775, 0.9991664010110769, 1.1447097560674194e-05)


def _bandit_loss_body(score_hbm, idx_hbm, rew_hbm, out_hbm,
                      idx_v, vals_v, rew_v, out_v,
                      sem_i0, sem_i1, sem_i2,
                      sem_g0, sem_g1, sem_g2, sem_r, sem_o):
    cid = lax.axis_index("c")
    sid = lax.axis_index("s")
    sem_i = [sem_i0, sem_i1, sem_i2]
    sem_g = [sem_g0, sem_g1, sem_g2]
    k = [jnp.float32(v) for v in _P5]

    def run(chunks, base):
        n = sum(chunks)
        offs = [sum(chunks[:c]) for c in range(len(chunks))]
        cp_i = [
            pltpu.async_copy(
                idx_hbm.at[pl.ds(base + offs[c], chunks[c])],
                idx_v.at[pl.ds(offs[c], chunks[c])],
                sem_i[c],
            )
            for c in range(len(chunks))
        ]
        cp_r = pltpu.async_copy(rew_hbm.at[pl.ds(base, n)], rew_v.at[pl.ds(0, n)], sem_r)
        cp_g = []
        for c in range(len(chunks)):
            cp_i[c].wait()
            cp_g.append(pltpu.async_copy(
                score_hbm.at[idx_v.at[pl.ds(offs[c], chunks[c])]],
                vals_v.at[pl.ds(offs[c], chunks[c])],
                sem_g[c],
            ))
        cp_r.wait()
        cp_o = []
        for c in range(len(chunks)):
            cp_g[c].wait()
            for i in range(chunks[c] // L):
                s = pl.ds(offs[c] + i * L, L)
                x = vals_v[s]
                r = rew_v[s]
                u = jnp.exp(-jnp.abs(x))
                p = ((((k[0] * u + k[1]) * u + k[2]) * u + k[3]) * u + k[4]) * u + k[5]
                m = jnp.minimum(x, jnp.float32(0.0))
                out_v[s] = (p - m) * r
            cp_o.append(pltpu.async_copy(
                out_v.at[pl.ds(offs[c], chunks[c])],
                out_hbm.at[pl.ds(base + offs[c], chunks[c])],
                sem_o,
            ))
        for cp in cp_o:
            cp.wait()

    @pl.when(cid == HI_CORE)
    def _():
        run(CH_HI, sid * N_HI)

    @pl.when(cid != HI_CORE)
    def _():
        run(CH_LO, NS * N_HI + sid * N_LO)


@jax.jit
def _bandit_loss(score, idx, rewards):
    mesh = plsc.VectorSubcoreMesh(core_axis_name="c", subcore_axis_name="s")
    return pl.kernel(
        _bandit_loss_body,
        out_type=jax.ShapeDtypeStruct((B,), jnp.float32),
        mesh=mesh,
        scratch_types=[
            pltpu.VMEM((N_HI,), jnp.int32),
            pltpu.VMEM((N_HI,), jnp.float32),
            pltpu.VMEM((N_HI,), jnp.float32),
            pltpu.VMEM((N_HI,), jnp.float32),
        ] + [pltpu.SemaphoreType.DMA] * 8,
    )(score, idx, rewards)


def kernel(score, chosen_action, rewards):
    idx = chosen_action.astype(jnp.int32)
    return _bandit_loss(score, idx, rewards)
